# Initial kernel scaffold; baseline (speedup 1.0000x reference)
import jax
import jax.numpy as jnp
from jax.experimental import pallas as pl


def kernel(feature_map, rois):
    raise NotImplementedError("write your pallas kernel here")


# speedup vs baseline: 14.1763x; 14.1763x over previous
import jax
import jax.numpy as jnp
from jax import lax
from jax.experimental import pallas as pl
from jax.experimental.pallas import tpu as pltpu
from jax.experimental.pallas import tpu_sc as plsc

_POOL = 7
_B, _H, _W, _C = 4, 50, 50, 256
_R = 300
_NC, _NS = 2, 16
_NW = _NC * _NS
_NROI = _B * _R
_RPW = -(-_NROI // _NW)
_NROI_PAD = _NW * _RPW
_WPAD = 32
_MAXR = 9
_NCH = _C // 16


def _roi_meta(rois):
    r = rois.reshape(_NROI, 4)
    h0 = (_H * r[:, 0]).astype(jnp.int32)
    w0 = (_W * r[:, 1]).astype(jnp.int32)
    h1 = (_H * r[:, 2]).astype(jnp.int32)
    w1 = (_W * r[:, 3]).astype(jnp.int32)
    rh = h1 - h0
    rw = w1 - w0
    hstep = rh // _POOL
    wstep = rw // _POOL
    b = jnp.repeat(jnp.arange(_B, dtype=jnp.int32), _R)
    wbase = jnp.minimum(w0, _W - _WPAD)
    dw = w0 - wbase
    z = jnp.zeros_like(h0)
    meta = jnp.stack([b, h0, wbase, dw, hstep, wstep, rh, rw] + [z] * 8, axis=1)
    pad = jnp.tile(meta[:1], (_NROI_PAD - _NROI, 1))
    return jnp.concatenate([meta, pad], axis=0).reshape(_NW, _RPW * 16)


def _sc_body(fm, meta_hbm, out_hbm, meta_v, rowsbuf, outbuf):
    c = lax.axis_index("c")
    s = lax.axis_index("s")
    wid = c * _NS + s
    pltpu.sync_copy(meta_hbm.at[wid], meta_v)

    def do_roi(t, carry):
        mv = meta_v[pl.ds(t * 16, 16)]
        b = mv[0]
        h0 = mv[1]
        wbase = mv[2]
        dw = mv[3]
        hstep = mv[4]
        wstep = mv[5]
        rh = mv[6]
        rw = mv[7]

        for i in range(_POOL):
            nr = jnp.where(i == _POOL - 1, rh - (_POOL - 1) * hstep, hstep)
            y0 = h0 + i * hstep

            def dma_row(rr, cc):
                pltpu.sync_copy(fm.at[b, y0 + rr, pl.ds(wbase, _WPAD), :],
                                rowsbuf.at[rr])
                return cc

            lax.fori_loop(0, nr, dma_row, 0)

            for j in range(_POOL):
                nc = jnp.where(j == _POOL - 1, rw - (_POOL - 1) * wstep, wstep)
                c0 = dw + j * wstep

                def col_loop(w, accs):
                    def row_loop(rr, a):
                        return tuple(
                            jnp.maximum(a[ch],
                                        rowsbuf[rr, c0 + w, pl.ds(ch * 16, 16)])
                            for ch in range(_NCH))
                    return lax.fori_loop(0, nr, row_loop, accs)

                init = tuple(jnp.full((16,), -jnp.inf, jnp.float32)
                             for _ in range(_NCH))
                accs = lax.fori_loop(0, nc, col_loop, init)
                for ch in range(_NCH):
                    outbuf[i, j, pl.ds(ch * 16, 16)] = accs[ch]

        pltpu.sync_copy(outbuf, out_hbm.at[wid * _RPW + t])
        return carry

    lax.fori_loop(0, _RPW, do_roi, 0)


def kernel(feature_map, rois):
    meta = _roi_meta(rois)
    mesh = plsc.VectorSubcoreMesh(core_axis_name="c", subcore_axis_name="s")
    run = pl.kernel(
        _sc_body,
        mesh=mesh,
        out_type=jax.ShapeDtypeStruct((_NROI_PAD, _POOL, _POOL, _C),
                                      jnp.float32),
        scratch_types=[
            pltpu.VMEM((_RPW * 16,), jnp.int32),
            pltpu.VMEM((_MAXR, _WPAD, _C), jnp.float32),
            pltpu.VMEM((_POOL, _POOL, _C), jnp.float32),
        ],
        compiler_params=pltpu.CompilerParams(use_tc_tiling_on_sc=False),
    )
    out = run(feature_map, meta)
    return out[:_NROI].reshape(_B, _R, _POOL, _POOL, _C)


# speedup vs baseline: 25.5209x; 1.8003x over previous
import jax
import jax.numpy as jnp
from jax import lax
from jax.experimental import pallas as pl
from jax.experimental.pallas import tpu as pltpu
from jax.experimental.pallas import tpu_sc as plsc

_POOL = 7
_B, _H, _W, _C = 4, 50, 50, 256
_R = 300
_NC, _NS = 2, 16
_NW = _NC * _NS
_NROI = _B * _R
_RPW = -(-_NROI // _NW)
_NROI_PAD = _NW * _RPW
_WPAD = 28
_CHROWS = 4
_NCH = _C // 16


def _roi_meta(rois):
    r = rois.reshape(_NROI, 4)
    h0 = (_H * r[:, 0]).astype(jnp.int32)
    w0 = (_W * r[:, 1]).astype(jnp.int32)
    h1 = (_H * r[:, 2]).astype(jnp.int32)
    w1 = (_W * r[:, 3]).astype(jnp.int32)
    rh = h1 - h0
    rw = w1 - w0
    hstep = rh // _POOL
    wstep = rw // _POOL
    b = jnp.repeat(jnp.arange(_B, dtype=jnp.int32), _R)
    wbase = jnp.minimum(w0, _W - _WPAD)
    dw = w0 - wbase
    nr6 = rh - (_POOL - 1) * hstep
    nchunks = (_POOL - 1) + (nr6 + _CHROWS - 1) // _CHROWS
    meta = jnp.stack([b, h0, wbase, dw, hstep, wstep, rh, rw, nr6, nchunks]
                     + [jnp.zeros_like(h0)] * 6, axis=1)
    pad = jnp.tile(meta[:1], (_NROI_PAD - _NROI, 1))
    return jnp.concatenate([meta, pad], axis=0).reshape(_NW, _RPW * 16)


def _sc_body(fm, meta_hbm, out_hbm, meta_v, rowbuf, outbuf, dsem, osem):
    c = lax.axis_index("c")
    s = lax.axis_index("s")
    wid = c * _NS + s
    pltpu.sync_copy(meta_hbm.at[wid], meta_v)

    def do_roi(t, carry):
        mv = meta_v[pl.ds(t * 16, 16)]
        b = mv[0]
        h0 = mv[1]
        wbase = mv[2]
        dw = mv[3]
        hstep = mv[4]
        wstep = mv[5]
        rw = mv[7]
        nr6 = mv[8]
        nchunks = mv[9]
        pout = lax.rem(t, 2)

        def chunk_geom(k):
            bin_i = jnp.minimum(k, _POOL - 1)
            p = jnp.maximum(k - (_POOL - 1), 0)
            y0 = h0 + bin_i * hstep + p * _CHROWS
            nrk = jnp.where(k >= _POOL - 1,
                            jnp.minimum(_CHROWS, nr6 - p * _CHROWS), hstep)
            return bin_i, y0, nrk

        def row_slice(y):
            return fm.at[b, y, pl.ds(wbase, _WPAD), :]

        def issue_chunk(k):
            par = lax.rem(k, 2)
            _, y0, nrk = chunk_geom(k)

            def issue_row(r, cc):
                pltpu.async_copy(row_slice(y0 + r), rowbuf.at[par, r],
                                 dsem.at[par])
                return cc

            lax.fori_loop(0, nrk, issue_row, 0)

        issue_chunk(jnp.int32(0))

        @pl.when(t >= 2)
        def _():
            pltpu.make_async_copy(outbuf.at[pout],
                                  out_hbm.at[wid * _RPW + t - 2],
                                  osem.at[pout]).wait()

        def do_chunk(k, cc):
            par = lax.rem(k, 2)
            bin_i, y0, nrk = chunk_geom(k)

            @pl.when(k + 1 < nchunks)
            def _():
                issue_chunk(k + 1)

            def drain_row(r, cc2):
                pltpu.make_async_copy(row_slice(y0 + r), rowbuf.at[par, r],
                                      dsem.at[par]).wait()
                return cc2

            lax.fori_loop(0, nrk, drain_row, 0)

            fresh = k <= _POOL - 1
            for j in range(_POOL):
                nc = jnp.where(j == _POOL - 1, rw - (_POOL - 1) * wstep, wstep)
                c0 = dw + j * wstep

                def col_loop(w, accs):
                    def row_loop(r, a):
                        return tuple(
                            jnp.maximum(a[ch],
                                        rowbuf[par, r, c0 + w,
                                               pl.ds(ch * 16, 16)])
                            for ch in range(_NCH))
                    return lax.fori_loop(0, nrk, row_loop, accs)

                ninf = jnp.full((16,), -jnp.inf, jnp.float32)
                init = tuple(
                    jnp.where(fresh, ninf,
                              outbuf[pout, bin_i, j, pl.ds(ch * 16, 16)])
                    for ch in range(_NCH))
                accs = lax.fori_loop(0, nc, col_loop, init)
                for ch in range(_NCH):
                    outbuf[pout, bin_i, j, pl.ds(ch * 16, 16)] = accs[ch]
            return cc

        lax.fori_loop(0, nchunks, do_chunk, 0)

        pltpu.async_copy(outbuf.at[pout], out_hbm.at[wid * _RPW + t],
                         osem.at[pout])
        return carry

    lax.fori_loop(0, _RPW, do_roi, 0)

    def final_drain(t, cc):
        pout = lax.rem(t, 2)
        pltpu.make_async_copy(outbuf.at[pout], out_hbm.at[wid * _RPW + t],
                              osem.at[pout]).wait()
        return cc

    lax.fori_loop(_RPW - 2, _RPW, final_drain, 0)


def kernel(feature_map, rois):
    meta = _roi_meta(rois)
    mesh = plsc.VectorSubcoreMesh(core_axis_name="c", subcore_axis_name="s")
    run = pl.kernel(
        _sc_body,
        mesh=mesh,
        out_type=jax.ShapeDtypeStruct((_NROI_PAD, _POOL, _POOL, _C),
                                      jnp.float32),
        scratch_types=[
            pltpu.VMEM((_RPW * 16,), jnp.int32),
            pltpu.VMEM((2, _CHROWS, _WPAD, _C), jnp.float32),
            pltpu.VMEM((2, _POOL, _POOL, _C), jnp.float32),
            pltpu.SemaphoreType.DMA((2,)),
            pltpu.SemaphoreType.DMA((2,)),
        ],
        compiler_params=pltpu.CompilerParams(use_tc_tiling_on_sc=False),
    )
    out = run(feature_map, meta)
    return out[:_NROI].reshape(_B, _R, _POOL, _POOL, _C)


# speedup vs baseline: 28.8592x; 1.1308x over previous
import jax
import jax.numpy as jnp
from jax import lax
from jax.experimental import pallas as pl
from jax.experimental.pallas import tpu as pltpu
from jax.experimental.pallas import tpu_sc as plsc

_POOL = 7
_B, _H, _W, _C = 4, 50, 50, 256
_R = 300
_NC, _NS = 2, 16
_NW = _NC * _NS
_NROI = _B * _R
_RPW = -(-_NROI // _NW)
_NROI_PAD = _NW * _RPW
_WPAD = 28
_CHROWS = 4
_NCH = _C // 16
_NSLOT = 3


def _roi_meta(rois):
    r = rois.reshape(_NROI, 4)
    h0 = (_H * r[:, 0]).astype(jnp.int32)
    w0 = (_W * r[:, 1]).astype(jnp.int32)
    h1 = (_H * r[:, 2]).astype(jnp.int32)
    w1 = (_W * r[:, 3]).astype(jnp.int32)
    rh = h1 - h0
    rw = w1 - w0
    hstep = rh // _POOL
    wstep = rw // _POOL
    b = jnp.repeat(jnp.arange(_B, dtype=jnp.int32), _R)
    wbase = jnp.minimum(w0, _W - _WPAD)
    dw = w0 - wbase
    nr6 = rh - (_POOL - 1) * hstep
    nchunks = (_POOL - 1) + (nr6 + _CHROWS - 1) // _CHROWS
    meta = jnp.stack([b, h0, wbase, dw, hstep, wstep, rh, rw, nr6, nchunks]
                     + [jnp.zeros_like(h0)] * 6, axis=1)
    pad = jnp.tile(meta[:1], (_NROI_PAD - _NROI, 1))
    return jnp.concatenate([meta, pad], axis=0).reshape(_NW, _RPW * 16)


def _sc_body(fm, meta_hbm, out_hbm, meta_v, rowbuf, outbuf, dsem, osem):
    c = lax.axis_index("c")
    s = lax.axis_index("s")
    wid = c * _NS + s
    pltpu.sync_copy(meta_hbm.at[wid], meta_v)
    ninf = jnp.full((16,), -jnp.inf, jnp.float32)

    def chunk_rows(tt, k):
        mv = meta_v[pl.ds(tt * 16, 16)]
        bin_i = jnp.minimum(k, _POOL - 1)
        p = jnp.maximum(k - (_POOL - 1), 0)
        y0 = mv[1] + bin_i * mv[4] + p * _CHROWS
        nrk = jnp.where(k >= _POOL - 1,
                        jnp.minimum(_CHROWS, mv[8] - p * _CHROWS), mv[4])
        return mv[0], mv[2], y0, nrk

    def issue_chunk(tt, k, slot):
        b, wbase, y0, nrk = chunk_rows(tt, k)

        def issue_row(r, cc):
            pltpu.async_copy(fm.at[b, y0 + r, pl.ds(wbase, _WPAD), :],
                             rowbuf.at[slot, r], dsem.at[slot])
            return cc

        lax.fori_loop(0, nrk, issue_row, 0)

    def drain_chunk(tt, k, slot):
        b, wbase, y0, nrk = chunk_rows(tt, k)

        def drain_row(r, cc):
            pltpu.make_async_copy(fm.at[b, y0 + r, pl.ds(wbase, _WPAD), :],
                                  rowbuf.at[slot, r], dsem.at[slot]).wait()
            return cc

        lax.fori_loop(0, nrk, drain_row, 0)

    def issue_ahead(t, kk, cbase, nchunks):
        slot = lax.rem(cbase + kk, _NSLOT)

        @pl.when(kk < nchunks)
        def _():
            issue_chunk(t, kk, slot)

        @pl.when((kk >= nchunks) & (t + 1 < _RPW))
        def _():
            issue_chunk(t + 1, kk - nchunks, slot)

    issue_chunk(jnp.int32(0), jnp.int32(0), jnp.int32(0))
    issue_chunk(jnp.int32(0), jnp.int32(1), jnp.int32(1))

    def bin_block(par, nrk, j, wstep, rw, dw, init):
        nc = jnp.where(j == _POOL - 1, rw - (_POOL - 1) * wstep, wstep)
        c0 = dw + j * wstep

        def col_loop(w, accs):
            def row_loop(r, a):
                return tuple(
                    jnp.maximum(a[ch],
                                rowbuf[par, r, c0 + w, pl.ds(ch * 16, 16)])
                    for ch in range(_NCH))
            return lax.fori_loop(0, nrk, row_loop, accs)

        return lax.fori_loop(0, nc, col_loop, init)

    def do_roi(t, cbase):
        mv = meta_v[pl.ds(t * 16, 16)]
        dw = mv[3]
        hstep = mv[4]
        wstep = mv[5]
        rw = mv[7]
        nchunks = mv[9]
        pout = lax.rem(t, 2)

        @pl.when(t >= 2)
        def _():
            pltpu.make_async_copy(outbuf.at[pout],
                                  out_hbm.at[wid * _RPW + t - 2],
                                  osem.at[pout]).wait()

        def main_chunk(k, cc):
            slot = lax.rem(cbase + k, _NSLOT)
            issue_ahead(t, k + 2, cbase, nchunks)
            drain_chunk(t, k, slot)
            for j in range(_POOL):
                init = tuple(ninf for _ in range(_NCH))
                accs = bin_block(slot, hstep, j, wstep, rw, dw, init)
                for ch in range(_NCH):
                    outbuf[pout, k, j, pl.ds(ch * 16, 16)] = accs[ch]
            return cc

        lax.fori_loop(0, _POOL - 1, main_chunk, 0)

        for j in range(_POOL):
            for ch in range(_NCH):
                outbuf[pout, _POOL - 1, j, pl.ds(ch * 16, 16)] = ninf

        def piece_chunk(k, cc):
            slot = lax.rem(cbase + k, _NSLOT)
            issue_ahead(t, k + 2, cbase, nchunks)
            _, _, _, nrk = chunk_rows(t, k)
            drain_chunk(t, k, slot)
            for j in range(_POOL):
                init = tuple(outbuf[pout, _POOL - 1, j, pl.ds(ch * 16, 16)]
                             for ch in range(_NCH))
                accs = bin_block(slot, nrk, j, wstep, rw, dw, init)
                for ch in range(_NCH):
                    outbuf[pout, _POOL - 1, j, pl.ds(ch * 16, 16)] = accs[ch]
            return cc

        lax.fori_loop(_POOL - 1, nchunks, piece_chunk, 0)

        pltpu.async_copy(outbuf.at[pout], out_hbm.at[wid * _RPW + t],
                         osem.at[pout])
        return cbase + nchunks

    lax.fori_loop(0, _RPW, do_roi, jnp.int32(0))

    def final_drain(t, cc):
        pout = lax.rem(t, 2)
        pltpu.make_async_copy(outbuf.at[pout], out_hbm.at[wid * _RPW + t],
                              osem.at[pout]).wait()
        return cc

    lax.fori_loop(_RPW - 2, _RPW, final_drain, 0)


def kernel(feature_map, rois):
    meta = _roi_meta(rois)
    mesh = plsc.VectorSubcoreMesh(core_axis_name="c", subcore_axis_name="s")
    run = pl.kernel(
        _sc_body,
        mesh=mesh,
        out_type=jax.ShapeDtypeStruct((_NROI_PAD, _POOL, _POOL, _C),
                                      jnp.float32),
        scratch_types=[
            pltpu.VMEM((_RPW * 16,), jnp.int32),
            pltpu.VMEM((_NSLOT, _CHROWS, _WPAD, _C), jnp.float32),
            pltpu.VMEM((2, _POOL, _POOL, _C), jnp.float32),
            pltpu.SemaphoreType.DMA((_NSLOT,)),
            pltpu.SemaphoreType.DMA((2,)),
        ],
        compiler_params=pltpu.CompilerParams(use_tc_tiling_on_sc=False),
    )
    out = run(feature_map, meta)
    return out[:_NROI].reshape(_B, _R, _POOL, _POOL, _C)
